# Initial kernel scaffold; baseline (speedup 1.0000x reference)
import jax
import jax.numpy as jnp
from jax.experimental import pallas as pl


def kernel(x, edge_index, edge_type, w1, root1, b1, w2, root2, b2, lw, lb):
    raise NotImplementedError("write your pallas kernel here")


# speedup vs baseline: 6.2235x; 6.2235x over previous
import functools

import jax
import jax.numpy as jnp
from jax import lax
from jax.experimental import pallas as pl
from jax.experimental.pallas import tpu as pltpu
from jax.experimental.pallas import tpu_sc as plsc

_N = 10000
_E = 320000
_F = 128
_R = 3
_D = 64
_K = 128
_NS = 16
_NC = 2
_CH = 160
_EPT = _CH * _K
_EPAD = _EPT * _NS
_IB = 8
_NBLK = _CH // _IB
_ACC = 30016
_ZR = _ACC // _NS
_CW = 16
_B = 1000
_GRID = _N // _B

def _sc_accum_body(tbl, gidx, sidx, out, gblk_v, sblk_v, rows_v, acc_sh, sem):
    c = lax.axis_index("c")
    s = lax.axis_index("s")
    z16 = jnp.zeros((16,), jnp.float32)

    def _zb(i, carry):
        rows_v[i // (_D // 16), pl.ds((i % (_D // 16)) * 16, 16)] = z16
        return carry

    lax.fori_loop(0, _K * (_D // 16), _zb, None)
    base = s * _ZR
    nfull, rem = divmod(_ZR, _K)
    for b in range(nfull):
        pltpu.sync_copy(rows_v, acc_sh.at[pl.ds(base + b * _K, _K)])
    if rem:
        pltpu.sync_copy(rows_v.at[pl.ds(0, rem)],
                        acc_sh.at[pl.ds(base + nfull * _K, rem)])
    plsc.subcore_barrier()

    def _blk(i, carry):
        pltpu.sync_copy(gidx.at[c, s, pl.ds(i * _IB, _IB)], gblk_v)
        pltpu.sync_copy(sidx.at[s, pl.ds(i * _IB, _IB)], sblk_v)

        def _chunk(j, carry2):
            pltpu.async_copy(tbl.at[gblk_v.at[j]], rows_v, sem).wait()
            pltpu.sync_copy(rows_v, acc_sh.at[sblk_v.at[j]], add=True)
            return carry2

        lax.fori_loop(0, _IB, _chunk, None)
        return carry

    lax.fori_loop(0, _NBLK, _blk, None)
    plsc.subcore_barrier()
    pltpu.sync_copy(acc_sh.at[pl.ds(base, _ZR)], out.at[c, pl.ds(base, _ZR)])


def _sc_count_body(sidx, out, sidx_v, ones_v, zbuf_v, acc_sh):
    c = lax.axis_index("c")
    s = lax.axis_index("s")
    z16 = jnp.zeros((16,), jnp.float32)
    o16 = jnp.ones((16,), jnp.float32)

    def _zb(i, carry):
        zbuf_v[i, pl.ds(0, 16)] = z16
        return carry

    lax.fori_loop(0, _ZR, _zb, None)

    def _ob(i, carry):
        ones_v[i, pl.ds(0, 16)] = o16
        return carry

    lax.fori_loop(0, _K, _ob, None)
    base = s * _ZR
    pltpu.sync_copy(zbuf_v, acc_sh.at[pl.ds(base, _ZR)])
    plsc.subcore_barrier()

    pltpu.sync_copy(sidx.at[s], sidx_v)

    def _chunk(j, carry):
        pltpu.sync_copy(ones_v, acc_sh.at[sidx_v.at[j]], add=True)
        return carry

    lax.fori_loop(0, _CH, _chunk, None)
    plsc.subcore_barrier()
    pltpu.sync_copy(acc_sh.at[pl.ds(base, _ZR)], out.at[c, pl.ds(base, _ZR)])


def _sc_scratch():
    return dict(
        accum=[
            pltpu.VMEM((_IB, _K), jnp.int32),
            pltpu.VMEM((_IB, _K), jnp.int32),
            pltpu.VMEM((_K, _D), jnp.float32),
            pltpu.VMEM_SHARED((_ACC, _D), jnp.float32),
            pltpu.SemaphoreType.DMA,
        ],
        count=[
            pltpu.VMEM((_CH, _K), jnp.int32),
            pltpu.VMEM((_K, _CW), jnp.float32),
            pltpu.VMEM((_ZR, _CW), jnp.float32),
            pltpu.VMEM_SHARED((_ACC, _CW), jnp.float32),
        ],
    )


@functools.cache
def _sc_kernels():
    mesh = plsc.VectorSubcoreMesh(core_axis_name="c", subcore_axis_name="s")
    params = pltpu.CompilerParams(use_tc_tiling_on_sc=False)
    sc = _sc_scratch()
    accum = pl.kernel(
        _sc_accum_body,
        mesh=mesh,
        compiler_params=params,
        out_type=jax.ShapeDtypeStruct((_NC, _ACC, _D), jnp.float32),
        scratch_types=sc["accum"],
    )
    count = pl.kernel(
        _sc_count_body,
        mesh=mesh,
        compiler_params=params,
        out_type=jax.ShapeDtypeStruct((_NC, _ACC, _CW), jnp.float32),
        scratch_types=sc["count"],
    )
    return accum, count


def _tc1_body(x_ref, a_ref, cnt_ref, w_ref, root_ref, b_ref, out_ref):
    acc = jnp.dot(x_ref[...], root_ref[...],
                  preferred_element_type=jnp.float32) + b_ref[...]
    sc = 1.0 / jnp.maximum(cnt_ref[...], 1.0)
    for r in range(_R):
        acc = acc + jnp.dot(a_ref[r], w_ref[r],
                            preferred_element_type=jnp.float32) * sc[:, r:r + 1]
    h = jnp.maximum(acc, 0.0)
    out_ref[0] = h[:, :_D]
    out_ref[1] = h[:, _D:]


def _tc2_body(h_ref, a_ref, cnt_ref, w_ref, root_ref, b_ref, lw_ref, lb_ref,
              out_ref):
    h = jnp.concatenate([h_ref[0], h_ref[1]], axis=1)
    acc = jnp.dot(h, root_ref[...],
                  preferred_element_type=jnp.float32) + b_ref[...]
    sc = 1.0 / jnp.maximum(cnt_ref[...], 1.0)
    for r in range(_R):
        acc = acc + jnp.dot(a_ref[r], w_ref[r],
                            preferred_element_type=jnp.float32) * sc[:, r:r + 1]
    g = jnp.maximum(acc, 0.0)
    logits = jnp.dot(g, lw_ref[...],
                     preferred_element_type=jnp.float32) + lb_ref[...]
    m = jnp.max(logits, axis=1, keepdims=True)
    lse = m + jnp.log(jnp.sum(jnp.exp(logits - m), axis=1, keepdims=True))
    out_ref[...] = logits - lse


_tc1_in_specs = [
    pl.BlockSpec((_B, _F), lambda i: (i, 0)),
    pl.BlockSpec((_R, _B, _F), lambda i: (0, i, 0)),
    pl.BlockSpec((_B, _R), lambda i: (i, 0)),
    pl.BlockSpec((_R, _F, _F), lambda i: (0, 0, 0)),
    pl.BlockSpec((_F, _F), lambda i: (0, 0)),
    pl.BlockSpec((1, _F), lambda i: (0, 0)),
]
_tc1_out_specs = pl.BlockSpec((2, _B, _D), lambda i: (0, i, 0))
_tc2_in_specs = [
    pl.BlockSpec((2, _B, _D), lambda i: (0, i, 0)),
    pl.BlockSpec((_R, _B, _F), lambda i: (0, i, 0)),
    pl.BlockSpec((_B, _R), lambda i: (i, 0)),
    pl.BlockSpec((_R, _F, _F), lambda i: (0, 0, 0)),
    pl.BlockSpec((_F, _F), lambda i: (0, 0)),
    pl.BlockSpec((1, _F), lambda i: (0, 0)),
    pl.BlockSpec((_F, 2), lambda i: (0, 0)),
    pl.BlockSpec((1, 2), lambda i: (0, 0)),
]
_tc2_out_specs = pl.BlockSpec((_B, 2), lambda i: (i, 0))

_tc1 = pl.pallas_call(
    _tc1_body,
    grid=(_GRID,),
    in_specs=_tc1_in_specs,
    out_specs=_tc1_out_specs,
    out_shape=jax.ShapeDtypeStruct((2, _N, _D), jnp.float32),
)

_tc2 = pl.pallas_call(
    _tc2_body,
    grid=(_GRID,),
    in_specs=_tc2_in_specs,
    out_specs=_tc2_out_specs,
    out_shape=jax.ShapeDtypeStruct((_N, 2), jnp.float32),
)


def kernel(x, edge_index, edge_type, w1, root1, b1, w2, root2, b2, lw, lb):
    src = edge_index[0].astype(jnp.int32)
    dst = edge_index[1].astype(jnp.int32)
    et = edge_type.astype(jnp.int32)
    pad = _EPAD - _E

    sidx = et * _N + dst
    sidx = jnp.concatenate(
        [sidx, jnp.full((pad,), _R * _N, jnp.int32)]).reshape(_NS, _CH, _K)
    g0 = jnp.concatenate([src, jnp.zeros((pad,), jnp.int32)])
    gidx = jnp.stack([g0, g0 + _N]).reshape(_NC, _NS, _CH, _K)

    _sc_accum, _sc_count = _sc_kernels()
    cnt_out = _sc_count(sidx)
    cntT = cnt_out[0, :_R * _N, 0].reshape(_R, _N).T

    xT2 = jnp.concatenate([x[:, :_D], x[:, _D:]], axis=0)
    a1 = _sc_accum(xT2, gidx, sidx)
    a1 = jnp.concatenate([a1[0, :_R * _N], a1[1, :_R * _N]],
                         axis=1).reshape(_R, _N, _F)
    h2l = _tc1(x, a1, cntT, w1, root1, b1.reshape(1, _F))

    a2 = _sc_accum(h2l.reshape(_NC * _N, _D), gidx, sidx)
    a2 = jnp.concatenate([a2[0, :_R * _N], a2[1, :_R * _N]],
                         axis=1).reshape(_R, _N, _F)
    return _tc2(h2l, a2, cntT, w2, root2, b2.reshape(1, _F), lw,
                lb.reshape(1, 2))


# speedup vs baseline: 7.9314x; 1.2744x over previous
import functools

import jax
import jax.numpy as jnp
from jax import lax
from jax.experimental import pallas as pl
from jax.experimental.pallas import tpu as pltpu
from jax.experimental.pallas import tpu_sc as plsc

_N = 10000
_E = 320000
_F = 128
_R = 3
_D = 64
_K = 64
_NS = 16
_NC = 2
_CH = 320
_EPT = _CH * _K
_EPAD = _EPT * _NS
_IB = 20
_NBLK = _CH // _IB
_ACC = 30016
_ZR = _ACC // _NS
_CW = 16
_B = 1000
_GRID = _N // _B

def _sc_accum_body(tbl, gidx, sidx, out, gblk_v, sblk_v, rows0_v, rows1_v,
                   acc_sh, sem0, sem1):
    c = lax.axis_index("c")
    s = lax.axis_index("s")
    z16 = jnp.zeros((16,), jnp.float32)

    def _zb(i, carry):
        rows0_v[i // (_D // 16), pl.ds((i % (_D // 16)) * 16, 16)] = z16
        return carry

    lax.fori_loop(0, _K * (_D // 16), _zb, None)
    base = s * _ZR
    nfull, rem = divmod(_ZR, _K)
    for b in range(nfull):
        pltpu.sync_copy(rows0_v, acc_sh.at[pl.ds(base + b * _K, _K)])
    if rem:
        pltpu.sync_copy(rows0_v.at[pl.ds(0, rem)],
                        acc_sh.at[pl.ds(base + nfull * _K, rem)])
    plsc.subcore_barrier()

    bufs = (rows0_v, rows1_v)
    sems = (sem0, sem1)

    def _blk(i, carry):
        pltpu.sync_copy(gidx.at[c, s, pl.ds(i * _IB, _IB)], gblk_v)
        pltpu.sync_copy(sidx.at[s, pl.ds(i * _IB, _IB)], sblk_v)
        hs = [
            pltpu.async_copy(tbl.at[gblk_v.at[0]], bufs[0], sems[0]),
            pltpu.async_copy(tbl.at[gblk_v.at[1]], bufs[1], sems[1]),
        ]
        for j in range(2, _IB + 2):
            p = j % 2
            hs[p].wait()
            pltpu.sync_copy(bufs[p], acc_sh.at[sblk_v.at[j - 2]], add=True)
            if j < _IB:
                hs[p] = pltpu.async_copy(tbl.at[gblk_v.at[j]], bufs[p],
                                         sems[p])
        return carry

    lax.fori_loop(0, _NBLK, _blk, None)
    plsc.subcore_barrier()
    pltpu.sync_copy(acc_sh.at[pl.ds(base, _ZR)], out.at[c, pl.ds(base, _ZR)])


def _sc_count_body(sidx, out, sidx_v, ones_v, zbuf_v, acc_sh):
    c = lax.axis_index("c")
    s = lax.axis_index("s")
    z16 = jnp.zeros((16,), jnp.float32)
    o16 = jnp.ones((16,), jnp.float32)

    def _zb(i, carry):
        zbuf_v[i, pl.ds(0, 16)] = z16
        return carry

    lax.fori_loop(0, _ZR, _zb, None)

    def _ob(i, carry):
        ones_v[i, pl.ds(0, 16)] = o16
        return carry

    lax.fori_loop(0, _K, _ob, None)
    base = s * _ZR
    pltpu.sync_copy(zbuf_v, acc_sh.at[pl.ds(base, _ZR)])
    plsc.subcore_barrier()

    pltpu.sync_copy(sidx.at[s], sidx_v)

    def _chunk(j, carry):
        pltpu.sync_copy(ones_v, acc_sh.at[sidx_v.at[j]], add=True)
        return carry

    lax.fori_loop(0, _CH, _chunk, None)
    plsc.subcore_barrier()
    pltpu.sync_copy(acc_sh.at[pl.ds(base, _ZR)], out.at[c, pl.ds(base, _ZR)])


def _sc_scratch():
    return dict(
        accum=[
            pltpu.VMEM((_IB, _K), jnp.int32),
            pltpu.VMEM((_IB, _K), jnp.int32),
            pltpu.VMEM((_K, _D), jnp.float32),
            pltpu.VMEM((_K, _D), jnp.float32),
            pltpu.VMEM_SHARED((_ACC, _D), jnp.float32),
            pltpu.SemaphoreType.DMA,
            pltpu.SemaphoreType.DMA,
        ],
        count=[
            pltpu.VMEM((_CH, _K), jnp.int32),
            pltpu.VMEM((_K, _CW), jnp.float32),
            pltpu.VMEM((_ZR, _CW), jnp.float32),
            pltpu.VMEM_SHARED((_ACC, _CW), jnp.float32),
        ],
    )


@functools.cache
def _sc_kernels():
    mesh = plsc.VectorSubcoreMesh(core_axis_name="c", subcore_axis_name="s")
    params = pltpu.CompilerParams(use_tc_tiling_on_sc=False)
    sc = _sc_scratch()
    accum = pl.kernel(
        _sc_accum_body,
        mesh=mesh,
        compiler_params=params,
        out_type=jax.ShapeDtypeStruct((_NC, _ACC, _D), jnp.float32),
        scratch_types=sc["accum"],
    )
    count = pl.kernel(
        _sc_count_body,
        mesh=mesh,
        compiler_params=params,
        out_type=jax.ShapeDtypeStruct((_NC, _ACC, _CW), jnp.float32),
        scratch_types=sc["count"],
    )
    return accum, count


def _tc1_body(x_ref, a_ref, cnt_ref, w_ref, root_ref, b_ref, out_ref):
    acc = jnp.dot(x_ref[...], root_ref[...],
                  preferred_element_type=jnp.float32) + b_ref[...]
    sc = 1.0 / jnp.maximum(cnt_ref[...], 1.0)
    for r in range(_R):
        acc = acc + jnp.dot(a_ref[r], w_ref[r],
                            preferred_element_type=jnp.float32) * sc[:, r:r + 1]
    h = jnp.maximum(acc, 0.0)
    out_ref[0] = h[:, :_D]
    out_ref[1] = h[:, _D:]


def _tc2_body(h_ref, a_ref, cnt_ref, w_ref, root_ref, b_ref, lw_ref, lb_ref,
              out_ref):
    h = jnp.concatenate([h_ref[0], h_ref[1]], axis=1)
    acc = jnp.dot(h, root_ref[...],
                  preferred_element_type=jnp.float32) + b_ref[...]
    sc = 1.0 / jnp.maximum(cnt_ref[...], 1.0)
    for r in range(_R):
        acc = acc + jnp.dot(a_ref[r], w_ref[r],
                            preferred_element_type=jnp.float32) * sc[:, r:r + 1]
    g = jnp.maximum(acc, 0.0)
    logits = jnp.dot(g, lw_ref[...],
                     preferred_element_type=jnp.float32) + lb_ref[...]
    m = jnp.max(logits, axis=1, keepdims=True)
    lse = m + jnp.log(jnp.sum(jnp.exp(logits - m), axis=1, keepdims=True))
    out_ref[...] = logits - lse


_tc1_in_specs = [
    pl.BlockSpec((_B, _F), lambda i: (i, 0)),
    pl.BlockSpec((_R, _B, _F), lambda i: (0, i, 0)),
    pl.BlockSpec((_B, _R), lambda i: (i, 0)),
    pl.BlockSpec((_R, _F, _F), lambda i: (0, 0, 0)),
    pl.BlockSpec((_F, _F), lambda i: (0, 0)),
    pl.BlockSpec((1, _F), lambda i: (0, 0)),
]
_tc1_out_specs = pl.BlockSpec((2, _B, _D), lambda i: (0, i, 0))
_tc2_in_specs = [
    pl.BlockSpec((2, _B, _D), lambda i: (0, i, 0)),
    pl.BlockSpec((_R, _B, _F), lambda i: (0, i, 0)),
    pl.BlockSpec((_B, _R), lambda i: (i, 0)),
    pl.BlockSpec((_R, _F, _F), lambda i: (0, 0, 0)),
    pl.BlockSpec((_F, _F), lambda i: (0, 0)),
    pl.BlockSpec((1, _F), lambda i: (0, 0)),
    pl.BlockSpec((_F, 2), lambda i: (0, 0)),
    pl.BlockSpec((1, 2), lambda i: (0, 0)),
]
_tc2_out_specs = pl.BlockSpec((_B, 2), lambda i: (i, 0))

_tc1 = pl.pallas_call(
    _tc1_body,
    grid=(_GRID,),
    in_specs=_tc1_in_specs,
    out_specs=_tc1_out_specs,
    out_shape=jax.ShapeDtypeStruct((2, _N, _D), jnp.float32),
)

_tc2 = pl.pallas_call(
    _tc2_body,
    grid=(_GRID,),
    in_specs=_tc2_in_specs,
    out_specs=_tc2_out_specs,
    out_shape=jax.ShapeDtypeStruct((_N, 2), jnp.float32),
)


def kernel(x, edge_index, edge_type, w1, root1, b1, w2, root2, b2, lw, lb):
    src = edge_index[0].astype(jnp.int32)
    dst = edge_index[1].astype(jnp.int32)
    et = edge_type.astype(jnp.int32)
    pad = _EPAD - _E

    sidx = et * _N + dst
    sidx = jnp.concatenate(
        [sidx, jnp.full((pad,), _R * _N, jnp.int32)]).reshape(_NS, _CH, _K)
    g0 = jnp.concatenate([src, jnp.zeros((pad,), jnp.int32)])
    gidx = jnp.stack([g0, g0 + _N]).reshape(_NC, _NS, _CH, _K)

    _sc_accum, _sc_count = _sc_kernels()
    cnt_out = _sc_count(sidx)
    cntT = cnt_out[0, :_R * _N, 0].reshape(_R, _N).T

    xT2 = jnp.concatenate([x[:, :_D], x[:, _D:]], axis=0)
    a1 = _sc_accum(xT2, gidx, sidx)
    a1 = jnp.concatenate([a1[0, :_R * _N], a1[1, :_R * _N]],
                         axis=1).reshape(_R, _N, _F)
    h2l = _tc1(x, a1, cntT, w1, root1, b1.reshape(1, _F))

    a2 = _sc_accum(h2l.reshape(_NC * _N, _D), gidx, sidx)
    a2 = jnp.concatenate([a2[0, :_R * _N], a2[1, :_R * _N]],
                         axis=1).reshape(_R, _N, _F)
    return _tc2(h2l, a2, cntT, w2, root2, b2.reshape(1, _F), lw,
                lb.reshape(1, 2))
